# baseline (speedup 1.0000x reference)
import functools

import jax
import jax.numpy as jnp
from jax import lax
from jax.experimental import pallas as pl
from jax.experimental.pallas import tpu as pltpu
from jax.experimental.pallas import tpu_sc as plsc

NUM_CLASSES = 21
THRESHOLD = 0.5
NEGPOS_RATIO = 3
VAR0, VAR1 = 0.1, 0.2

B = 32
P = 6375
O = 16
L = 16
PPAD = 6400
NG = PPAD // L
CHUNK_P = 640
NFULL = P // CHUNK_P
TAIL_P = P - NFULL * CHUNK_P

_LN2 = 0.6931471805599453
_SQRT2 = 1.4142135623730951


def _hsum(v):
  s = v[0]
  for i in range(1, L):
    s = s + v[i]
  return s


def _hmax(v):
  s = v[0]
  for i in range(1, L):
    s = jnp.maximum(s, v[i])
  return s


def _hmin(v):
  s = v[0]
  for i in range(1, L):
    s = jnp.minimum(s, v[i])
  return s


def _ln(x):
  b = plsc.bitcast(x, jnp.int32)
  e = lax.shift_right_logical(b, 23) - 127
  m = plsc.bitcast((b & 0x007FFFFF) | 0x3F800000, jnp.float32)
  big = m > _SQRT2
  m = jnp.where(big, m * 0.5, m)
  e = jnp.where(big, e + 1, e)
  z = (m - 1.0) / (m + 1.0)
  z2 = z * z
  p = 2.0 + z2 * (2.0 / 3.0 + z2 * (2.0 / 5.0 + z2 * (2.0 / 7.0 + z2 * (2.0 / 9.0))))
  return e.astype(jnp.float32) * _LN2 + z * p


def _body(conf_h, loc_h, pri_h, tgt_h, out_h,
          pv, lv, tv, bto_r, bti_r, ct_r, vv_r, buf, buf2, pbuf, lbuf,
          hist, res, sem, sem2):
  img = lax.axis_index("s") * 2 + lax.axis_index("c")
  iota = lax.iota(jnp.int32, L)

  half = jnp.full((L,), 0.5)
  for c in range(4):
    pv[pl.ds(c * PPAD + PPAD - 2 * L, L)] = half
    pv[pl.ds(c * PPAD + PPAD - L, L)] = half
    lv[pl.ds(c * PPAD + PPAD - 2 * L, L)] = half
    lv[pl.ds(c * PPAD + PPAD - L, L)] = half
  for ci in range(NFULL + 1):
    pstart = ci * CHUNK_P
    cnt = CHUNK_P if ci < NFULL else TAIL_P
    pb = pbuf if ci < NFULL else pbuf.at[pl.ds(0, TAIL_P)]
    lb = lbuf if ci < NFULL else lbuf.at[pl.ds(0, TAIL_P)]
    pltpu.sync_copy(pri_h.at[pl.ds(pstart, cnt)], pb)
    pltpu.sync_copy(loc_h.at[img, pl.ds(pstart, cnt)], lb)
    ngr = (cnt + L - 1) // L

    def unpack(g, _, pstart=pstart):
      ip = g * L + iota
      for c in range(4):
        cf = jnp.full((L,), c, jnp.int32)
        pv[pl.ds(c * PPAD + pstart + g * L, L)] = plsc.load_gather(
            pbuf, [ip, cf])
        lv[pl.ds(c * PPAD + pstart + g * L, L)] = plsc.load_gather(
            lbuf, [ip, cf])
      return 0

    lax.fori_loop(0, ngr, unpack, 0)
  pltpu.sync_copy(tgt_h.at[img], tv)

  def tcol(idx, c):
    return plsc.load_gather(tv, [idx, jnp.full((L,), c, jnp.int32)])

  r0 = tcol(iota, 0)
  r1 = tcol(iota, 1)
  r2 = tcol(iota, 2)
  r3 = tcol(iota, 3)
  t_x0 = [jnp.full((L,), r0[t]) for t in range(O)]
  t_y0 = [jnp.full((L,), r1[t]) for t in range(O)]
  t_x1 = [jnp.full((L,), r2[t]) for t in range(O)]
  t_y1 = [jnp.full((L,), r3[t]) for t in range(O)]
  t_ar = [(t_x1[t] - t_x0[t]) * (t_y1[t] - t_y0[t]) for t in range(O)]

  def g1(g, carry):
    bv, bi = carry
    base = g * L
    pidx = base + iota
    valid = pidx < P
    px = pv[pl.ds(base, L)]
    py = pv[pl.ds(PPAD + base, L)]
    pw = pv[pl.ds(2 * PPAD + base, L)]
    ph = pv[pl.ds(3 * PPAD + base, L)]
    x0 = px - pw * 0.5
    x1 = px + pw * 0.5
    y0 = py - ph * 0.5
    y1 = py + ph * 0.5
    area_p = pw * ph
    bto_g = jnp.full((L,), -1.0)
    bti_g = jnp.zeros((L,), jnp.int32)
    nbv = []
    nbi = []
    for t in range(O):
      ix0 = jnp.maximum(x0, t_x0[t])
      ix1 = jnp.minimum(x1, t_x1[t])
      iy0 = jnp.maximum(y0, t_y0[t])
      iy1 = jnp.minimum(y1, t_y1[t])
      iw = jnp.maximum(ix1 - ix0, 0.0)
      ih = jnp.maximum(iy1 - iy0, 0.0)
      inter = iw * ih
      iou = inter / (t_ar[t] + area_p - inter)
      up = iou > bto_g
      bto_g = jnp.where(up, iou, bto_g)
      bti_g = jnp.where(up, t, bti_g)
      iou_m = jnp.where(valid, iou, -1.0)
      upt = iou_m > bv[t]
      nbv.append(jnp.where(upt, iou_m, bv[t]))
      nbi.append(jnp.where(upt, pidx, bi[t]))
    bto_r[pl.ds(base, L)] = jnp.where(valid, bto_g, 0.0)
    bti_r[pl.ds(base, L)] = bti_g
    return tuple(nbv), tuple(nbi)

  init = (tuple(jnp.full((L,), -2.0) for _ in range(O)),
          tuple(jnp.zeros((L,), jnp.int32) for _ in range(O)))
  bvf, bif = lax.fori_loop(0, NG, g1, init)

  bpiv = jnp.zeros((L,), jnp.int32)
  for t in range(O):
    m = _hmax(bvf[t])
    cand = jnp.where(bvf[t] == m, bif[t], jnp.int32(P))
    bpiv = jnp.where(iota == t, jnp.full((L,), _hmin(cand)), bpiv)

  plsc.store_scatter(bto_r, [bpiv], jnp.full((L,), 2.0))
  plsc.store_scatter(bti_r, [bpiv], iota)

  def g3(g, carry):
    ll, npv = carry
    base = g * L
    btog = bto_r[pl.ds(base, L)]
    btig = bti_r[pl.ds(base, L)]
    pos = btog >= THRESHOLD
    lab = tcol(btig, 4)
    ct = jnp.where(pos, lab.astype(jnp.int32) + 1, 0)
    ct_r[pl.ds(base, L)] = ct
    m0 = tcol(btig, 0)
    m1 = tcol(btig, 1)
    m2 = tcol(btig, 2)
    m3 = tcol(btig, 3)
    px = pv[pl.ds(base, L)]
    py = pv[pl.ds(PPAD + base, L)]
    pw = pv[pl.ds(2 * PPAD + base, L)]
    ph = pv[pl.ds(3 * PPAD + base, L)]
    g0 = ((m0 + m2) * 0.5 - px) / (pw * VAR0)
    g1v = ((m1 + m3) * 0.5 - py) / (ph * VAR0)
    g2 = _ln((m2 - m0) / pw) * (1.0 / VAR1)
    g3v = _ln((m3 - m1) / ph) * (1.0 / VAR1)
    acc = jnp.zeros((L,))
    for c, gc in enumerate((g0, g1v, g2, g3v)):
      d = lv[pl.ds(c * PPAD + base, L)] - gc
      ad = jnp.abs(d)
      acc = acc + jnp.where(ad < 1.0, 0.5 * d * d, ad - 0.5)
    ll = ll + jnp.where(pos, acc, 0.0)
    npv = npv + jnp.where(pos, 1, 0).astype(jnp.int32)
    return ll, npv

  ll, npv = lax.fori_loop(0, NG, g3, (jnp.zeros((L,)), jnp.zeros((L,), jnp.int32)))
  loss_l = _hsum(ll)
  npos = _hsum(npv)

  sp_acc = jnp.zeros((L,))
  bufs = (buf, buf2)
  sems = (sem, sem2)

  def start_chunk(ci):
    pstart = ci * CHUNK_P
    cnt = CHUNK_P if ci < NFULL else TAIL_P
    b = bufs[ci % 2]
    dst = b if ci < NFULL else b.at[pl.ds(0, TAIL_P)]
    return pltpu.async_copy(
        conf_h.at[img, pl.ds(pstart, cnt)], dst, sems[ci % 2])

  descs = [start_chunk(0)]
  for ci in range(NFULL + 1):
    if ci + 1 <= NFULL:
      descs.append(start_chunk(ci + 1))
    descs[ci].wait()
    pstart = ci * CHUNK_P
    cnt = CHUNK_P if ci < NFULL else TAIL_P
    cb = bufs[ci % 2]
    ngr = (cnt + L - 1) // L

    def g4(g, acc, pstart=pstart, cb=cb):
      base_l = g * L
      base_g = pstart + base_l
      pidx = base_g + iota
      valid = pidx < P
      ip = base_l + iota
      xs = [plsc.load_gather(cb, [ip, jnp.full((L,), j, jnp.int32)])
            for j in range(NUM_CLASSES)]
      m = xs[0]
      for j in range(1, NUM_CLASSES):
        m = jnp.maximum(m, xs[j])
      s = jnp.zeros((L,))
      for j in range(NUM_CLASSES):
        s = s + jnp.exp(xs[j] - m)
      lse = _ln(s) + m
      ctg = ct_r[pl.ds(base_g, L)]
      tgt = plsc.load_gather(cb, [ip, ctg])
      ce = lse - tgt
      pos = ctg > 0
      acc = acc + jnp.where(pos & valid, ce, 0.0)
      vv_r[pl.ds(base_g, L)] = jnp.where(pos | (~valid), 0.0, ce)
      return acc

    sp_acc = lax.fori_loop(0, ngr, g4, sp_acc)
  vv_r[pl.ds(NFULL * CHUNK_P + ((TAIL_P + L - 1) // L) * L,
             PPAD - NFULL * CHUNK_P - ((TAIL_P + L - 1) // L) * L)] = (
      jnp.zeros((PPAD - NFULL * CHUNK_P - ((TAIL_P + L - 1) // L) * L,)))

  k = jnp.minimum(jnp.int32(NEGPOS_RATIO) * npos, jnp.int32(P - 1))

  ones = jnp.ones((L,), jnp.int32)
  prefix = jnp.int32(0)
  kk = k
  for shift, width in ((22, 9), (13, 9), (4, 9), (0, 4)):
    nbins = 1 << width
    ngrp = nbins // L

    def zh(g, _):
      hist[pl.ds(g * L, L)] = jnp.zeros((L,), jnp.int32)
      return 0

    lax.fori_loop(0, ngrp, zh, 0)
    pref_v = jnp.full((L,), prefix)
    lo_mask = jnp.int32(nbins - 1)

    def hp(g, _, shift=shift, width=width, pref_v=pref_v, lo_mask=lo_mask):
      vb = plsc.bitcast(vv_r[pl.ds(g * L, L)], jnp.int32)
      m = lax.shift_right_logical(vb, shift + width) == pref_v
      b = lax.shift_right_logical(vb, shift) & lo_mask
      plsc.addupdate_scatter(hist, [b], ones, mask=m)
      return 0

    lax.fori_loop(0, NG, hp, 0)

    def sg(i, carry, ngrp=ngrp):
      cum, fbin, krem = carry
      gi = ngrp - 1 - i
      hv = hist[pl.ds(gi * L, L)]
      for lane in range(L - 1, -1, -1):
        c = hv[lane]
        ncum = cum + c
        hit = (ncum >= kk) & (cum < kk)
        fbin = jnp.where(hit, gi * L + lane, fbin)
        krem = jnp.where(hit, kk - cum, krem)
        cum = ncum
      return cum, fbin, krem

    _, fbin, krem = lax.fori_loop(
        0, ngrp, sg, (jnp.int32(0), jnp.int32(0), kk))
    prefix = (prefix << width) | fbin
    kk = krem

  bits = jnp.where(k > 0, prefix, jnp.int32(0x7F000000))
  bstar = jnp.full((L,), bits)

  def gf(g, carry):
    sh, ch = carry
    v = vv_r[pl.ds(g * L, L)]
    vb = plsc.bitcast(v, jnp.int32)
    gt = vb > bstar
    sh = sh + jnp.where(gt, v, 0.0)
    ch = ch + jnp.where(gt, 1, 0).astype(jnp.int32)
    return sh, ch

  sh, ch = lax.fori_loop(0, NG, gf, (jnp.zeros((L,)), jnp.zeros((L,), jnp.int32)))
  sum_hi = _hsum(sh)
  c_hi = _hsum(ch)
  tie_val = plsc.bitcast(bstar, jnp.float32)[0]
  loss_c = _hsum(sp_acc) + sum_hi + (k - c_hi).astype(jnp.float32) * tie_val

  outv = jnp.where(iota == 0, jnp.full((L,), loss_l),
                   jnp.where(iota == 1, jnp.full((L,), loss_c),
                             jnp.where(iota == 2,
                                       jnp.full((L,), npos.astype(jnp.float32)),
                                       0.0)))
  res[...] = outv
  pltpu.sync_copy(res, out_h.at[img])


@jax.jit
def _run(conf, loc, pri, tgt):
  mesh = plsc.VectorSubcoreMesh(core_axis_name="c", subcore_axis_name="s",
                                num_cores=2, num_subcores=16)
  f = pl.kernel(
      _body,
      out_type=jax.ShapeDtypeStruct((B, L), jnp.float32),
      mesh=mesh,
      compiler_params=pltpu.CompilerParams(needs_layout_passes=False,
                                           use_tc_tiling_on_sc=False),
      scratch_types=[
          pltpu.VMEM((4 * PPAD,), jnp.float32),
          pltpu.VMEM((4 * PPAD,), jnp.float32),
          pltpu.VMEM((O, 5), jnp.float32),
          pltpu.VMEM((PPAD,), jnp.float32),
          pltpu.VMEM((PPAD,), jnp.int32),
          pltpu.VMEM((PPAD,), jnp.int32),
          pltpu.VMEM((PPAD,), jnp.float32),
          pltpu.VMEM((CHUNK_P, NUM_CLASSES), jnp.float32),
          pltpu.VMEM((CHUNK_P, NUM_CLASSES), jnp.float32),
          pltpu.VMEM((CHUNK_P, 4), jnp.float32),
          pltpu.VMEM((CHUNK_P, 4), jnp.float32),
          pltpu.VMEM((512,), jnp.int32),
          pltpu.VMEM((L,), jnp.float32),
          pltpu.SemaphoreType.DMA,
          pltpu.SemaphoreType.DMA,
      ],
  )
  return f(conf, loc, pri, tgt)


def kernel(arm_loc_data, arm_conf_data, odm_loc_data, odm_conf_data,
           priors, targets):
  del odm_loc_data, odm_conf_data
  out = _run(arm_conf_data, arm_loc_data, priors, targets)
  loss_l = jnp.sum(out[:, 0])
  loss_c = jnp.sum(out[:, 1])
  n = jnp.sum(out[:, 2])
  return (loss_l / n, loss_c / n)


# speedup vs baseline: 1.4955x; 1.4955x over previous
import functools

import jax
import jax.numpy as jnp
from jax import lax
from jax.experimental import pallas as pl
from jax.experimental.pallas import tpu as pltpu
from jax.experimental.pallas import tpu_sc as plsc

NUM_CLASSES = 21
THRESHOLD = 0.5
NEGPOS_RATIO = 3
VAR0, VAR1 = 0.1, 0.2

B = 32
P = 6375
O = 16
L = 16
PPAD = 6400
NG = PPAD // L
CHUNK_P = 640
NFULL = P // CHUNK_P
TAIL_P = P - NFULL * CHUNK_P

_LN2 = 0.6931471805599453
_SQRT2 = 1.4142135623730951


def _hsum(v):
  s = v[0]
  for i in range(1, L):
    s = s + v[i]
  return s


def _hmax(v):
  s = v[0]
  for i in range(1, L):
    s = jnp.maximum(s, v[i])
  return s


def _hmin(v):
  s = v[0]
  for i in range(1, L):
    s = jnp.minimum(s, v[i])
  return s


def _maxtree(xs):
  xs = list(xs)
  while len(xs) > 1:
    nxt = [jnp.maximum(xs[i], xs[i + 1]) for i in range(0, len(xs) - 1, 2)]
    if len(xs) % 2:
      nxt.append(xs[-1])
    xs = nxt
  return xs[0]


def _sumtree(xs):
  xs = list(xs)
  while len(xs) > 1:
    nxt = [xs[i] + xs[i + 1] for i in range(0, len(xs) - 1, 2)]
    if len(xs) % 2:
      nxt.append(xs[-1])
    xs = nxt
  return xs[0]


def _ln(x):
  b = plsc.bitcast(x, jnp.int32)
  e = lax.shift_right_logical(b, 23) - 127
  m = plsc.bitcast((b & 0x007FFFFF) | 0x3F800000, jnp.float32)
  big = m > _SQRT2
  m = jnp.where(big, m * 0.5, m)
  e = jnp.where(big, e + 1, e)
  z = (m - 1.0) / (m + 1.0)
  z2 = z * z
  p = 2.0 + z2 * (2.0 / 3.0 + z2 * (2.0 / 5.0 + z2 * (2.0 / 7.0 + z2 * (2.0 / 9.0))))
  return e.astype(jnp.float32) * _LN2 + z * p


def _body(conf_h, loc_h, pri_h, tgt_h, out_h,
          pv, lv, tv, bto_r, bti_r, ct_r, vv_r, buf, buf2,
          hist, res, sem, sem2):
  img = lax.axis_index("s") * 2 + lax.axis_index("c")
  iota = lax.iota(jnp.int32, L)

  def stage_unpack(src_part, dstv):
    for roff, rcnt in ((0, 3200), (3200, P - 3200)):
      pltpu.sync_copy(src_part(roff * 4, rcnt * 4),
                      buf.at[pl.ds(0, rcnt * 4)])
      ngr = (rcnt + L - 1) // L

      def unpack(g, _, roff=roff):
        ip4 = (g * L + iota) * 4
        for c in range(4):
          dstv[pl.ds(c * PPAD + roff + g * L, L)] = plsc.load_gather(
              buf, [ip4 + c])
        return 0

      lax.fori_loop(0, ngr, unpack, 0)

  stage_unpack(lambda w, n: pri_h.at[pl.ds(w, n)], pv)
  stage_unpack(lambda w, n: loc_h.at[img, pl.ds(w, n)], lv)
  half = jnp.full((L,), 0.5)
  for c in range(4):
    padv = jnp.full((L,), -10.0 if c < 2 else 1e-4)
    for off in (P, PPAD - L):
      plsc.store_scatter(pv, [jnp.full((L,), c * PPAD + off) + iota], padv)
      plsc.store_scatter(lv, [jnp.full((L,), c * PPAD + off) + iota], half)
  pltpu.sync_copy(tgt_h.at[img], tv)

  def tcol(idx, c):
    return plsc.load_gather(tv, [idx * 5 + c])

  r0 = tcol(iota, 0)
  r1 = tcol(iota, 1)
  r2 = tcol(iota, 2)
  r3 = tcol(iota, 3)
  t_x0 = [jnp.full((L,), r0[t]) for t in range(O)]
  t_y0 = [jnp.full((L,), r1[t]) for t in range(O)]
  t_x1 = [jnp.full((L,), r2[t]) for t in range(O)]
  t_y1 = [jnp.full((L,), r3[t]) for t in range(O)]
  t_ar = [(t_x1[t] - t_x0[t]) * (t_y1[t] - t_y0[t]) for t in range(O)]

  def g1_sub(base, bv, bi, nbv, nbi):
    pidx = base + iota
    px = pv[pl.ds(base, L)]
    py = pv[pl.ds(PPAD + base, L)]
    pw = pv[pl.ds(2 * PPAD + base, L)]
    ph = pv[pl.ds(3 * PPAD + base, L)]
    x0 = px - pw * 0.5
    x1 = px + pw * 0.5
    y0 = py - ph * 0.5
    y1 = py + ph * 0.5
    area_p = pw * ph
    bto_g = jnp.full((L,), -1.0)
    bti_g = jnp.zeros((L,), jnp.int32)
    for t in range(O):
      ix0 = jnp.maximum(x0, t_x0[t])
      ix1 = jnp.minimum(x1, t_x1[t])
      iy0 = jnp.maximum(y0, t_y0[t])
      iy1 = jnp.minimum(y1, t_y1[t])
      iw = jnp.maximum(ix1 - ix0, 0.0)
      ih = jnp.maximum(iy1 - iy0, 0.0)
      inter = iw * ih
      iou = inter / (t_ar[t] + area_p - inter)
      up = iou > bto_g
      bto_g = jnp.where(up, iou, bto_g)
      bti_g = jnp.where(up, t, bti_g)
      upt = iou > bv[t]
      nbv[t] = jnp.where(upt, iou, bv[t])
      nbi[t] = jnp.where(upt, pidx, bi[t])
    bto_r[pl.ds(base, L)] = bto_g
    bti_r[pl.ds(base, L)] = bti_g

  def g1(g, carry):
    bv, bi = carry
    nbv = list(bv)
    nbi = list(bi)
    g1_sub(g * (2 * L), nbv, nbi, nbv, nbi)
    g1_sub(g * (2 * L) + L, tuple(nbv), tuple(nbi), nbv, nbi)
    return tuple(nbv), tuple(nbi)

  init = (tuple(jnp.full((L,), -2.0) for _ in range(O)),
          tuple(jnp.zeros((L,), jnp.int32) for _ in range(O)))
  bvf, bif = lax.fori_loop(0, NG // 2, g1, init)

  bpiv = jnp.zeros((L,), jnp.int32)
  for t in range(O):
    m = _hmax(bvf[t])
    cand = jnp.where(bvf[t] == m, bif[t], jnp.int32(P))
    bpiv = jnp.where(iota == t, jnp.full((L,), _hmin(cand)), bpiv)

  plsc.store_scatter(bto_r, [bpiv], jnp.full((L,), 2.0))
  plsc.store_scatter(bti_r, [bpiv], iota)

  def g3(g, carry):
    ll, npv = carry
    base = g * L
    btog = bto_r[pl.ds(base, L)]
    btig = bti_r[pl.ds(base, L)]
    pos = btog >= THRESHOLD
    lab = tcol(btig, 4)
    ct = jnp.where(pos, lab.astype(jnp.int32) + 1, 0)
    ct_r[pl.ds(base, L)] = ct
    m0 = tcol(btig, 0)
    m1 = tcol(btig, 1)
    m2 = tcol(btig, 2)
    m3 = tcol(btig, 3)
    px = pv[pl.ds(base, L)]
    py = pv[pl.ds(PPAD + base, L)]
    pw = pv[pl.ds(2 * PPAD + base, L)]
    ph = pv[pl.ds(3 * PPAD + base, L)]
    g0 = ((m0 + m2) * 0.5 - px) / (pw * VAR0)
    g1v = ((m1 + m3) * 0.5 - py) / (ph * VAR0)
    g2 = _ln((m2 - m0) / pw) * (1.0 / VAR1)
    g3v = _ln((m3 - m1) / ph) * (1.0 / VAR1)
    acc = jnp.zeros((L,))
    for c, gc in enumerate((g0, g1v, g2, g3v)):
      d = lv[pl.ds(c * PPAD + base, L)] - gc
      ad = jnp.abs(d)
      acc = acc + jnp.where(ad < 1.0, 0.5 * d * d, ad - 0.5)
    ll = ll + jnp.where(pos, acc, 0.0)
    npv = npv + jnp.where(pos, 1, 0).astype(jnp.int32)
    return ll, npv

  ll, npv = lax.fori_loop(0, NG, g3, (jnp.zeros((L,)), jnp.zeros((L,), jnp.int32)))
  loss_l = _hsum(ll)
  npos = _hsum(npv)

  sp_acc = jnp.zeros((L,))
  bufs = (buf, buf2)
  sems = (sem, sem2)

  def start_chunk(ci):
    pstart = ci * CHUNK_P
    cnt = CHUNK_P if ci < NFULL else TAIL_P
    b = bufs[ci % 2]
    dst = b if ci < NFULL else b.at[pl.ds(0, TAIL_P * NUM_CLASSES)]
    return pltpu.async_copy(
        conf_h.at[img, pl.ds(pstart * NUM_CLASSES, cnt * NUM_CLASSES)], dst,
        sems[ci % 2])

  descs = [start_chunk(0)]
  for ci in range(NFULL + 1):
    if ci + 1 <= NFULL:
      descs.append(start_chunk(ci + 1))
    descs[ci].wait()
    pstart = ci * CHUNK_P
    cnt = CHUNK_P if ci < NFULL else TAIL_P
    cb = bufs[ci % 2]
    ngr = (cnt + L - 1) // L

    def ce_group(base_l, acc, pstart=pstart, cb=cb):
      base_g = pstart + base_l
      pidx = base_g + iota
      valid = pidx < P
      fidx = (base_l + iota) * NUM_CLASSES
      xs = [plsc.load_gather(cb, [fidx + j]) for j in range(NUM_CLASSES)]
      m = _maxtree(xs)
      lse = _ln(_sumtree([jnp.exp(x - m) for x in xs])) + m
      ctg = ct_r[pl.ds(base_g, L)]
      tgt = plsc.load_gather(cb, [fidx + ctg])
      ce = lse - tgt
      pos = ctg > 0
      acc = acc + jnp.where(pos & valid, ce, 0.0)
      vv_r[pl.ds(base_g, L)] = jnp.where(pos | (~valid), 0.0, ce)
      return acc

    if cnt == CHUNK_P:
      def g4(g, acc):
        acc = ce_group(g * (2 * L), acc)
        return ce_group(g * (2 * L) + L, acc)

      sp_acc = lax.fori_loop(0, ngr // 2, g4, sp_acc)
    else:
      def g4t(g, acc):
        return ce_group(g * L, acc)

      sp_acc = lax.fori_loop(0, ngr, g4t, sp_acc)
  vv_r[pl.ds(NFULL * CHUNK_P + ((TAIL_P + L - 1) // L) * L,
             PPAD - NFULL * CHUNK_P - ((TAIL_P + L - 1) // L) * L)] = (
      jnp.zeros((PPAD - NFULL * CHUNK_P - ((TAIL_P + L - 1) // L) * L,)))

  k = jnp.minimum(jnp.int32(NEGPOS_RATIO) * npos, jnp.int32(P - 1))

  ones = jnp.ones((L,), jnp.int32)
  prefix = jnp.int32(0)
  kk = k
  for shift, width in ((22, 9), (13, 9), (4, 9), (0, 4)):
    nbins = 1 << width
    ngrp = nbins // L

    def zh(g, _):
      hist[pl.ds(g * L, L)] = jnp.zeros((L,), jnp.int32)
      return 0

    lax.fori_loop(0, ngrp, zh, 0)
    pref_v = jnp.full((L,), prefix)
    lo_mask = jnp.int32(nbins - 1)

    def hp(g, _, shift=shift, width=width, pref_v=pref_v, lo_mask=lo_mask):
      vb = plsc.bitcast(vv_r[pl.ds(g * L, L)], jnp.int32)
      m = lax.shift_right_logical(vb, shift + width) == pref_v
      b = lax.shift_right_logical(vb, shift) & lo_mask
      plsc.addupdate_scatter(hist, [b], ones, mask=m)
      return 0

    lax.fori_loop(0, NG, hp, 0)

    def sg(i, carry, ngrp=ngrp):
      cum, fbin, krem = carry
      gi = ngrp - 1 - i
      hv = hist[pl.ds(gi * L, L)]
      for lane in range(L - 1, -1, -1):
        c = hv[lane]
        ncum = cum + c
        hit = (ncum >= kk) & (cum < kk)
        fbin = jnp.where(hit, gi * L + lane, fbin)
        krem = jnp.where(hit, kk - cum, krem)
        cum = ncum
      return cum, fbin, krem

    _, fbin, krem = lax.fori_loop(
        0, ngrp, sg, (jnp.int32(0), jnp.int32(0), kk))
    prefix = (prefix << width) | fbin
    kk = krem

  bits = jnp.where(k > 0, prefix, jnp.int32(0x7F000000))
  bstar = jnp.full((L,), bits)

  def gf(g, carry):
    sh, ch = carry
    v = vv_r[pl.ds(g * L, L)]
    vb = plsc.bitcast(v, jnp.int32)
    gt = vb > bstar
    sh = sh + jnp.where(gt, v, 0.0)
    ch = ch + jnp.where(gt, 1, 0).astype(jnp.int32)
    return sh, ch

  sh, ch = lax.fori_loop(0, NG, gf, (jnp.zeros((L,)), jnp.zeros((L,), jnp.int32)))
  sum_hi = _hsum(sh)
  c_hi = _hsum(ch)
  tie_val = plsc.bitcast(bstar, jnp.float32)[0]
  loss_c = _hsum(sp_acc) + sum_hi + (k - c_hi).astype(jnp.float32) * tie_val

  outv = jnp.where(iota == 0, jnp.full((L,), loss_l),
                   jnp.where(iota == 1, jnp.full((L,), loss_c),
                             jnp.where(iota == 2,
                                       jnp.full((L,), npos.astype(jnp.float32)),
                                       0.0)))
  res[...] = outv
  pltpu.sync_copy(res, out_h.at[img])


@jax.jit
def _run(conf, loc, pri, tgt):
  mesh = plsc.VectorSubcoreMesh(core_axis_name="c", subcore_axis_name="s",
                                num_cores=2, num_subcores=16)
  f = pl.kernel(
      _body,
      out_type=jax.ShapeDtypeStruct((B, L), jnp.float32),
      mesh=mesh,
      compiler_params=pltpu.CompilerParams(needs_layout_passes=False,
                                           use_tc_tiling_on_sc=False),
      scratch_types=[
          pltpu.VMEM((4 * PPAD,), jnp.float32),
          pltpu.VMEM((4 * PPAD,), jnp.float32),
          pltpu.VMEM((O * 5,), jnp.float32),
          pltpu.VMEM((PPAD,), jnp.float32),
          pltpu.VMEM((PPAD,), jnp.int32),
          pltpu.VMEM((PPAD,), jnp.int32),
          pltpu.VMEM((PPAD,), jnp.float32),
          pltpu.VMEM((CHUNK_P * NUM_CLASSES,), jnp.float32),
          pltpu.VMEM((CHUNK_P * NUM_CLASSES,), jnp.float32),
          pltpu.VMEM((512,), jnp.int32),
          pltpu.VMEM((L,), jnp.float32),
          pltpu.SemaphoreType.DMA,
          pltpu.SemaphoreType.DMA,
      ],
  )
  return f(conf, loc, pri, tgt)


def kernel(arm_loc_data, arm_conf_data, odm_loc_data, odm_conf_data,
           priors, targets):
  del odm_loc_data, odm_conf_data
  out = _run(arm_conf_data.reshape(B, P * NUM_CLASSES),
             arm_loc_data.reshape(B, P * 4),
             priors.reshape(P * 4),
             targets.reshape(B, O * 5))
  loss_l = jnp.sum(out[:, 0])
  loss_c = jnp.sum(out[:, 1])
  n = jnp.sum(out[:, 2])
  return (loss_l / n, loss_c / n)


# speedup vs baseline: 1.5613x; 1.0440x over previous
import functools

import jax
import jax.numpy as jnp
from jax import lax
from jax.experimental import pallas as pl
from jax.experimental.pallas import tpu as pltpu
from jax.experimental.pallas import tpu_sc as plsc

NUM_CLASSES = 21
THRESHOLD = 0.5
NEGPOS_RATIO = 3
VAR0, VAR1 = 0.1, 0.2

B = 32
P = 6375
O = 16
L = 16
PPAD = 6400
NG = PPAD // L
CHUNK_P = 640
NFULL = P // CHUNK_P
TAIL_P = P - NFULL * CHUNK_P

_LN2 = 0.6931471805599453
_SQRT2 = 1.4142135623730951


def _hsum(v):
  s = v[0]
  for i in range(1, L):
    s = s + v[i]
  return s


def _hmax(v):
  s = v[0]
  for i in range(1, L):
    s = jnp.maximum(s, v[i])
  return s


def _hmin(v):
  s = v[0]
  for i in range(1, L):
    s = jnp.minimum(s, v[i])
  return s


def _maxtree(xs):
  xs = list(xs)
  while len(xs) > 1:
    nxt = [jnp.maximum(xs[i], xs[i + 1]) for i in range(0, len(xs) - 1, 2)]
    if len(xs) % 2:
      nxt.append(xs[-1])
    xs = nxt
  return xs[0]


def _sumtree(xs):
  xs = list(xs)
  while len(xs) > 1:
    nxt = [xs[i] + xs[i + 1] for i in range(0, len(xs) - 1, 2)]
    if len(xs) % 2:
      nxt.append(xs[-1])
    xs = nxt
  return xs[0]


def _ln(x):
  b = plsc.bitcast(x, jnp.int32)
  e = lax.shift_right_logical(b, 23) - 127
  m = plsc.bitcast((b & 0x007FFFFF) | 0x3F800000, jnp.float32)
  big = m > _SQRT2
  m = jnp.where(big, m * 0.5, m)
  e = jnp.where(big, e + 1, e)
  z = (m - 1.0) / (m + 1.0)
  z2 = z * z
  p = 2.0 + z2 * (2.0 / 3.0 + z2 * (2.0 / 5.0 + z2 * (2.0 / 7.0 + z2 * (2.0 / 9.0))))
  return e.astype(jnp.float32) * _LN2 + z * p


def _body(conf_h, loc_h, pri_h, tgt_h, out_h,
          pv, lv, tv, bto_r, bti_r, ct_r, vv_r, buf, buf2,
          hist, res, sem, sem2):
  img = lax.axis_index("s") * 2 + lax.axis_index("c")
  iota = lax.iota(jnp.int32, L)

  def stage_unpack(src_part, dstv):
    for roff, rcnt in ((0, 3200), (3200, P - 3200)):
      pltpu.sync_copy(src_part(roff * 4, rcnt * 4),
                      buf.at[pl.ds(0, rcnt * 4)])
      ngr = (rcnt + L - 1) // L

      def unpack(g, _, roff=roff):
        ip4 = (g * L + iota) * 4
        for c in range(4):
          dstv[pl.ds(c * PPAD + roff + g * L, L)] = plsc.load_gather(
              buf, [ip4 + c])
        return 0

      lax.fori_loop(0, ngr, unpack, 0)

  stage_unpack(lambda w, n: pri_h.at[pl.ds(w, n)], pv)
  stage_unpack(lambda w, n: loc_h.at[img, pl.ds(w, n)], lv)
  half = jnp.full((L,), 0.5)
  for c in range(4):
    padv = jnp.full((L,), -10.0 if c < 2 else 1e-4)
    for off in (P, PPAD - L):
      plsc.store_scatter(pv, [jnp.full((L,), c * PPAD + off) + iota], padv)
      plsc.store_scatter(lv, [jnp.full((L,), c * PPAD + off) + iota], half)
  pltpu.sync_copy(tgt_h.at[img], tv)

  def tcol(idx, c):
    return plsc.load_gather(tv, [idx * 5 + c])

  r0 = tcol(iota, 0)
  r1 = tcol(iota, 1)
  r2 = tcol(iota, 2)
  r3 = tcol(iota, 3)
  t_x0 = [jnp.full((L,), r0[t]) for t in range(O)]
  t_y0 = [jnp.full((L,), r1[t]) for t in range(O)]
  t_x1 = [jnp.full((L,), r2[t]) for t in range(O)]
  t_y1 = [jnp.full((L,), r3[t]) for t in range(O)]
  t_ar = [(t_x1[t] - t_x0[t]) * (t_y1[t] - t_y0[t]) for t in range(O)]

  def g1_sweep(trange, first):
    def body(g, carry):
      bv, bi = carry
      base = g * L
      pidx = base + iota
      px = pv[pl.ds(base, L)]
      py = pv[pl.ds(PPAD + base, L)]
      pw = pv[pl.ds(2 * PPAD + base, L)]
      ph = pv[pl.ds(3 * PPAD + base, L)]
      x0 = px - pw * 0.5
      x1 = px + pw * 0.5
      y0 = py - ph * 0.5
      y1 = py + ph * 0.5
      area_p = pw * ph
      if first:
        bto_g = jnp.full((L,), -1.0)
        bti_g = jnp.zeros((L,), jnp.int32)
      else:
        bto_g = bto_r[pl.ds(base, L)]
        bti_g = bti_r[pl.ds(base, L)]
      nbv = list(bv)
      nbi = list(bi)
      for i, t in enumerate(trange):
        ix0 = jnp.maximum(x0, t_x0[t])
        ix1 = jnp.minimum(x1, t_x1[t])
        iy0 = jnp.maximum(y0, t_y0[t])
        iy1 = jnp.minimum(y1, t_y1[t])
        iw = jnp.maximum(ix1 - ix0, 0.0)
        ih = jnp.maximum(iy1 - iy0, 0.0)
        inter = iw * ih
        iou = inter / (t_ar[t] + area_p - inter)
        up = iou > bto_g
        bto_g = jnp.where(up, iou, bto_g)
        bti_g = jnp.where(up, t, bti_g)
        upt = iou > nbv[i]
        nbv[i] = jnp.where(upt, iou, nbv[i])
        nbi[i] = jnp.where(upt, pidx, nbi[i])
      bto_r[pl.ds(base, L)] = bto_g
      bti_r[pl.ds(base, L)] = bti_g
      return tuple(nbv), tuple(nbi)

    init = (tuple(jnp.full((L,), -2.0) for _ in trange),
            tuple(jnp.zeros((L,), jnp.int32) for _ in trange))
    return lax.fori_loop(0, NG, body, init)

  bvf0, bif0 = g1_sweep(range(0, 8), True)
  bvf1, bif1 = g1_sweep(range(8, O), False)
  bvf = tuple(bvf0) + tuple(bvf1)
  bif = tuple(bif0) + tuple(bif1)

  bpiv = jnp.zeros((L,), jnp.int32)
  for t in range(O):
    m = _hmax(bvf[t])
    cand = jnp.where(bvf[t] == m, bif[t], jnp.int32(P))
    bpiv = jnp.where(iota == t, jnp.full((L,), _hmin(cand)), bpiv)

  plsc.store_scatter(bto_r, [bpiv], jnp.full((L,), 2.0))
  plsc.store_scatter(bti_r, [bpiv], iota)

  def g3(g, carry):
    ll, npv = carry
    base = g * L
    btog = bto_r[pl.ds(base, L)]
    btig = bti_r[pl.ds(base, L)]
    pos = btog >= THRESHOLD
    lab = tcol(btig, 4)
    ct = jnp.where(pos, lab.astype(jnp.int32) + 1, 0)
    ct_r[pl.ds(base, L)] = ct
    m0 = tcol(btig, 0)
    m1 = tcol(btig, 1)
    m2 = tcol(btig, 2)
    m3 = tcol(btig, 3)
    px = pv[pl.ds(base, L)]
    py = pv[pl.ds(PPAD + base, L)]
    pw = pv[pl.ds(2 * PPAD + base, L)]
    ph = pv[pl.ds(3 * PPAD + base, L)]
    g0 = ((m0 + m2) * 0.5 - px) / (pw * VAR0)
    g1v = ((m1 + m3) * 0.5 - py) / (ph * VAR0)
    g2 = _ln((m2 - m0) / pw) * (1.0 / VAR1)
    g3v = _ln((m3 - m1) / ph) * (1.0 / VAR1)
    acc = jnp.zeros((L,))
    for c, gc in enumerate((g0, g1v, g2, g3v)):
      d = lv[pl.ds(c * PPAD + base, L)] - gc
      ad = jnp.abs(d)
      acc = acc + jnp.where(ad < 1.0, 0.5 * d * d, ad - 0.5)
    ll = ll + jnp.where(pos, acc, 0.0)
    npv = npv + jnp.where(pos, 1, 0).astype(jnp.int32)
    return ll, npv

  ll, npv = lax.fori_loop(0, NG, g3, (jnp.zeros((L,)), jnp.zeros((L,), jnp.int32)))
  loss_l = _hsum(ll)
  npos = _hsum(npv)

  sp_acc = jnp.zeros((L,))
  bufs = (buf, buf2)
  sems = (sem, sem2)

  def start_chunk(ci):
    pstart = ci * CHUNK_P
    cnt = CHUNK_P if ci < NFULL else TAIL_P
    b = bufs[ci % 2]
    dst = b if ci < NFULL else b.at[pl.ds(0, TAIL_P * NUM_CLASSES)]
    return pltpu.async_copy(
        conf_h.at[img, pl.ds(pstart * NUM_CLASSES, cnt * NUM_CLASSES)], dst,
        sems[ci % 2])

  descs = [start_chunk(0)]
  for ci in range(NFULL + 1):
    if ci + 1 <= NFULL:
      descs.append(start_chunk(ci + 1))
    descs[ci].wait()
    pstart = ci * CHUNK_P
    cnt = CHUNK_P if ci < NFULL else TAIL_P
    cb = bufs[ci % 2]
    ngr = (cnt + L - 1) // L

    def ce_group(base_l, acc, pstart=pstart, cb=cb):
      base_g = pstart + base_l
      pidx = base_g + iota
      valid = pidx < P
      fidx = (base_l + iota) * NUM_CLASSES
      xs = [plsc.load_gather(cb, [fidx + j]) for j in range(NUM_CLASSES)]
      m = _maxtree(xs)
      lse = _ln(_sumtree([jnp.exp(x - m) for x in xs])) + m
      ctg = ct_r[pl.ds(base_g, L)]
      tgt = plsc.load_gather(cb, [fidx + ctg])
      ce = lse - tgt
      pos = ctg > 0
      acc = acc + jnp.where(pos & valid, ce, 0.0)
      vv_r[pl.ds(base_g, L)] = jnp.where(pos | (~valid), 0.0, ce)
      return acc

    if cnt == CHUNK_P:
      def g4(g, acc):
        acc = ce_group(g * (2 * L), acc)
        return ce_group(g * (2 * L) + L, acc)

      sp_acc = lax.fori_loop(0, ngr // 2, g4, sp_acc)
    else:
      def g4t(g, acc):
        return ce_group(g * L, acc)

      sp_acc = lax.fori_loop(0, ngr, g4t, sp_acc)
  vv_r[pl.ds(NFULL * CHUNK_P + ((TAIL_P + L - 1) // L) * L,
             PPAD - NFULL * CHUNK_P - ((TAIL_P + L - 1) // L) * L)] = (
      jnp.zeros((PPAD - NFULL * CHUNK_P - ((TAIL_P + L - 1) // L) * L,)))

  k = jnp.minimum(jnp.int32(NEGPOS_RATIO) * npos, jnp.int32(P - 1))

  ones = jnp.ones((L,), jnp.int32)
  prefix = jnp.int32(0)
  kk = k
  for shift, width in ((22, 9), (13, 9), (4, 9), (0, 4)):
    nbins = 1 << width
    ngrp = nbins // L

    def zh(g, _):
      hist[pl.ds(g * L, L)] = jnp.zeros((L,), jnp.int32)
      return 0

    lax.fori_loop(0, ngrp, zh, 0)
    pref_v = jnp.full((L,), prefix)
    lo_mask = jnp.int32(nbins - 1)

    def hp(g, _, shift=shift, width=width, pref_v=pref_v, lo_mask=lo_mask):
      vb = plsc.bitcast(vv_r[pl.ds(g * L, L)], jnp.int32)
      m = lax.shift_right_logical(vb, shift + width) == pref_v
      b = lax.shift_right_logical(vb, shift) & lo_mask
      plsc.addupdate_scatter(hist, [b], ones, mask=m)
      return 0

    lax.fori_loop(0, NG, hp, 0)

    def sg(i, carry, ngrp=ngrp):
      cum, fbin, krem = carry
      gi = ngrp - 1 - i
      hv = hist[pl.ds(gi * L, L)]
      for lane in range(L - 1, -1, -1):
        c = hv[lane]
        ncum = cum + c
        hit = (ncum >= kk) & (cum < kk)
        fbin = jnp.where(hit, gi * L + lane, fbin)
        krem = jnp.where(hit, kk - cum, krem)
        cum = ncum
      return cum, fbin, krem

    _, fbin, krem = lax.fori_loop(
        0, ngrp, sg, (jnp.int32(0), jnp.int32(0), kk))
    prefix = (prefix << width) | fbin
    kk = krem

  bits = jnp.where(k > 0, prefix, jnp.int32(0x7F000000))
  bstar = jnp.full((L,), bits)

  def gf(g, carry):
    sh, ch = carry
    v = vv_r[pl.ds(g * L, L)]
    vb = plsc.bitcast(v, jnp.int32)
    gt = vb > bstar
    sh = sh + jnp.where(gt, v, 0.0)
    ch = ch + jnp.where(gt, 1, 0).astype(jnp.int32)
    return sh, ch

  sh, ch = lax.fori_loop(0, NG, gf, (jnp.zeros((L,)), jnp.zeros((L,), jnp.int32)))
  sum_hi = _hsum(sh)
  c_hi = _hsum(ch)
  tie_val = plsc.bitcast(bstar, jnp.float32)[0]
  loss_c = _hsum(sp_acc) + sum_hi + (k - c_hi).astype(jnp.float32) * tie_val

  outv = jnp.where(iota == 0, jnp.full((L,), loss_l),
                   jnp.where(iota == 1, jnp.full((L,), loss_c),
                             jnp.where(iota == 2,
                                       jnp.full((L,), npos.astype(jnp.float32)),
                                       0.0)))
  res[...] = outv
  pltpu.sync_copy(res, out_h.at[img])


@jax.jit
def _run(conf, loc, pri, tgt):
  mesh = plsc.VectorSubcoreMesh(core_axis_name="c", subcore_axis_name="s",
                                num_cores=2, num_subcores=16)
  f = pl.kernel(
      _body,
      out_type=jax.ShapeDtypeStruct((B, L), jnp.float32),
      mesh=mesh,
      compiler_params=pltpu.CompilerParams(needs_layout_passes=False,
                                           use_tc_tiling_on_sc=False),
      scratch_types=[
          pltpu.VMEM((4 * PPAD,), jnp.float32),
          pltpu.VMEM((4 * PPAD,), jnp.float32),
          pltpu.VMEM((O * 5,), jnp.float32),
          pltpu.VMEM((PPAD,), jnp.float32),
          pltpu.VMEM((PPAD,), jnp.int32),
          pltpu.VMEM((PPAD,), jnp.int32),
          pltpu.VMEM((PPAD,), jnp.float32),
          pltpu.VMEM((CHUNK_P * NUM_CLASSES,), jnp.float32),
          pltpu.VMEM((CHUNK_P * NUM_CLASSES,), jnp.float32),
          pltpu.VMEM((512,), jnp.int32),
          pltpu.VMEM((L,), jnp.float32),
          pltpu.SemaphoreType.DMA,
          pltpu.SemaphoreType.DMA,
      ],
  )
  return f(conf, loc, pri, tgt)


def kernel(arm_loc_data, arm_conf_data, odm_loc_data, odm_conf_data,
           priors, targets):
  del odm_loc_data, odm_conf_data
  one = lax.optimization_barrier(jnp.float32(1.0))
  out = _run(arm_conf_data.reshape(B, P * NUM_CLASSES) * one,
             arm_loc_data.reshape(B, P * 4) * one,
             priors.reshape(P * 4),
             targets.reshape(B, O * 5))
  loss_l = jnp.sum(out[:, 0])
  loss_c = jnp.sum(out[:, 1])
  n = jnp.sum(out[:, 2])
  return (loss_l / n, loss_c / n)


# speedup vs baseline: 1.5759x; 1.0094x over previous
import functools

import jax
import jax.numpy as jnp
from jax import lax
from jax.experimental import pallas as pl
from jax.experimental.pallas import tpu as pltpu
from jax.experimental.pallas import tpu_sc as plsc

NUM_CLASSES = 21
THRESHOLD = 0.5
NEGPOS_RATIO = 3
VAR0, VAR1 = 0.1, 0.2

B = 32
P = 6375
O = 16
L = 16
PPAD = 6400
NG = PPAD // L
CHUNK_P = 640
NFULL = P // CHUNK_P
TAIL_P = P - NFULL * CHUNK_P

_LN2 = 0.6931471805599453
_SQRT2 = 1.4142135623730951


def _hsum(v):
  s = v[0]
  for i in range(1, L):
    s = s + v[i]
  return s


def _hmax(v):
  s = v[0]
  for i in range(1, L):
    s = jnp.maximum(s, v[i])
  return s


def _hmin(v):
  s = v[0]
  for i in range(1, L):
    s = jnp.minimum(s, v[i])
  return s


def _maxtree(xs):
  xs = list(xs)
  while len(xs) > 1:
    nxt = [jnp.maximum(xs[i], xs[i + 1]) for i in range(0, len(xs) - 1, 2)]
    if len(xs) % 2:
      nxt.append(xs[-1])
    xs = nxt
  return xs[0]


def _sumtree(xs):
  xs = list(xs)
  while len(xs) > 1:
    nxt = [xs[i] + xs[i + 1] for i in range(0, len(xs) - 1, 2)]
    if len(xs) % 2:
      nxt.append(xs[-1])
    xs = nxt
  return xs[0]


def _ln(x):
  b = plsc.bitcast(x, jnp.int32)
  e = lax.shift_right_logical(b, 23) - 127
  m = plsc.bitcast((b & 0x007FFFFF) | 0x3F800000, jnp.float32)
  big = m > _SQRT2
  m = jnp.where(big, m * 0.5, m)
  e = jnp.where(big, e + 1, e)
  z = (m - 1.0) / (m + 1.0)
  z2 = z * z
  p = 2.0 + z2 * (2.0 / 3.0 + z2 * (2.0 / 5.0 + z2 * (2.0 / 7.0 + z2 * (2.0 / 9.0))))
  return e.astype(jnp.float32) * _LN2 + z * p


def _body(conf_h, loc_h, pri_h, tgt_h, out_h,
          pv, lv, tv, bto_r, bti_r, ct_r, vv_r, buf, buf2,
          hist, h2, res, sem, sem2):
  img = lax.axis_index("s") * 2 + lax.axis_index("c")
  iota = lax.iota(jnp.int32, L)

  def stage_unpack(src_part, dstv):
    for roff, rcnt in ((0, 3200), (3200, P - 3200)):
      pltpu.sync_copy(src_part(roff * 4, rcnt * 4),
                      buf.at[pl.ds(0, rcnt * 4)])
      ngr = (rcnt + L - 1) // L

      def unpack(g, _, roff=roff):
        ip4 = (g * L + iota) * 4
        for c in range(4):
          dstv[pl.ds(c * PPAD + roff + g * L, L)] = plsc.load_gather(
              buf, [ip4 + c])
        return 0

      lax.fori_loop(0, ngr, unpack, 0)

  stage_unpack(lambda w, n: pri_h.at[pl.ds(w, n)], pv)
  stage_unpack(lambda w, n: loc_h.at[img, pl.ds(w, n)], lv)
  half = jnp.full((L,), 0.5)
  for c in range(4):
    padv = jnp.full((L,), -10.0 if c < 2 else 1e-4)
    for off in (P, PPAD - L):
      plsc.store_scatter(pv, [jnp.full((L,), c * PPAD + off) + iota], padv)
      plsc.store_scatter(lv, [jnp.full((L,), c * PPAD + off) + iota], half)
  pltpu.sync_copy(tgt_h.at[img], tv)

  def tcol(idx, c):
    return plsc.load_gather(tv, [idx * 5 + c])

  r0 = tcol(iota, 0)
  r1 = tcol(iota, 1)
  r2 = tcol(iota, 2)
  r3 = tcol(iota, 3)
  t_x0 = [jnp.full((L,), r0[t]) for t in range(O)]
  t_y0 = [jnp.full((L,), r1[t]) for t in range(O)]
  t_x1 = [jnp.full((L,), r2[t]) for t in range(O)]
  t_y1 = [jnp.full((L,), r3[t]) for t in range(O)]
  t_ar = [(t_x1[t] - t_x0[t]) * (t_y1[t] - t_y0[t]) for t in range(O)]

  def g1_sweep(trange, first):
    def body(g, carry):
      bv, bi = carry
      base = g * L
      pidx = base + iota
      px = pv[pl.ds(base, L)]
      py = pv[pl.ds(PPAD + base, L)]
      pw = pv[pl.ds(2 * PPAD + base, L)]
      ph = pv[pl.ds(3 * PPAD + base, L)]
      x0 = px - pw * 0.5
      x1 = px + pw * 0.5
      y0 = py - ph * 0.5
      y1 = py + ph * 0.5
      area_p = pw * ph
      if first:
        bto_g = jnp.full((L,), -1.0)
        bti_g = jnp.zeros((L,), jnp.int32)
      else:
        bto_g = bto_r[pl.ds(base, L)]
        bti_g = bti_r[pl.ds(base, L)]
      nbv = list(bv)
      nbi = list(bi)
      for i, t in enumerate(trange):
        ix0 = jnp.maximum(x0, t_x0[t])
        ix1 = jnp.minimum(x1, t_x1[t])
        iy0 = jnp.maximum(y0, t_y0[t])
        iy1 = jnp.minimum(y1, t_y1[t])
        iw = jnp.maximum(ix1 - ix0, 0.0)
        ih = jnp.maximum(iy1 - iy0, 0.0)
        inter = iw * ih
        iou = inter / (t_ar[t] + area_p - inter)
        up = iou > bto_g
        bto_g = jnp.where(up, iou, bto_g)
        bti_g = jnp.where(up, t, bti_g)
        upt = iou > nbv[i]
        nbv[i] = jnp.where(upt, iou, nbv[i])
        nbi[i] = jnp.where(upt, pidx, nbi[i])
      bto_r[pl.ds(base, L)] = bto_g
      bti_r[pl.ds(base, L)] = bti_g
      return tuple(nbv), tuple(nbi)

    init = (tuple(jnp.full((L,), -2.0) for _ in trange),
            tuple(jnp.zeros((L,), jnp.int32) for _ in trange))
    return lax.fori_loop(0, NG, body, init)

  bvf0, bif0 = g1_sweep(range(0, 8), True)
  bvf1, bif1 = g1_sweep(range(8, O), False)
  bvf = tuple(bvf0) + tuple(bvf1)
  bif = tuple(bif0) + tuple(bif1)

  bpiv = jnp.zeros((L,), jnp.int32)
  for t in range(O):
    m = _hmax(bvf[t])
    cand = jnp.where(bvf[t] == m, bif[t], jnp.int32(P))
    bpiv = jnp.where(iota == t, jnp.full((L,), _hmin(cand)), bpiv)

  plsc.store_scatter(bto_r, [bpiv], jnp.full((L,), 2.0))
  plsc.store_scatter(bti_r, [bpiv], iota)

  def g3(g, carry):
    ll, npv = carry
    base = g * L
    btog = bto_r[pl.ds(base, L)]
    btig = bti_r[pl.ds(base, L)]
    pos = btog >= THRESHOLD
    lab = tcol(btig, 4)
    ct = jnp.where(pos, lab.astype(jnp.int32) + 1, 0)
    ct_r[pl.ds(base, L)] = ct
    m0 = tcol(btig, 0)
    m1 = tcol(btig, 1)
    m2 = tcol(btig, 2)
    m3 = tcol(btig, 3)
    px = pv[pl.ds(base, L)]
    py = pv[pl.ds(PPAD + base, L)]
    pw = pv[pl.ds(2 * PPAD + base, L)]
    ph = pv[pl.ds(3 * PPAD + base, L)]
    g0 = ((m0 + m2) * 0.5 - px) / (pw * VAR0)
    g1v = ((m1 + m3) * 0.5 - py) / (ph * VAR0)
    g2 = _ln((m2 - m0) / pw) * (1.0 / VAR1)
    g3v = _ln((m3 - m1) / ph) * (1.0 / VAR1)
    acc = jnp.zeros((L,))
    for c, gc in enumerate((g0, g1v, g2, g3v)):
      d = lv[pl.ds(c * PPAD + base, L)] - gc
      ad = jnp.abs(d)
      acc = acc + jnp.where(ad < 1.0, 0.5 * d * d, ad - 0.5)
    ll = ll + jnp.where(pos, acc, 0.0)
    npv = npv + jnp.where(pos, 1, 0).astype(jnp.int32)
    return ll, npv

  ll, npv = lax.fori_loop(0, NG, g3, (jnp.zeros((L,)), jnp.zeros((L,), jnp.int32)))
  loss_l = _hsum(ll)
  npos = _hsum(npv)

  sp_acc = jnp.zeros((L,))
  bufs = (buf, buf2)
  sems = (sem, sem2)

  def start_chunk(ci):
    pstart = ci * CHUNK_P
    cnt = CHUNK_P if ci < NFULL else TAIL_P
    b = bufs[ci % 2]
    dst = b if ci < NFULL else b.at[pl.ds(0, TAIL_P * NUM_CLASSES)]
    return pltpu.async_copy(
        conf_h.at[img, pl.ds(pstart * NUM_CLASSES, cnt * NUM_CLASSES)], dst,
        sems[ci % 2])

  descs = [start_chunk(0)]
  for ci in range(NFULL + 1):
    if ci + 1 <= NFULL:
      descs.append(start_chunk(ci + 1))
    descs[ci].wait()
    pstart = ci * CHUNK_P
    cnt = CHUNK_P if ci < NFULL else TAIL_P
    cb = bufs[ci % 2]
    ngr = (cnt + L - 1) // L

    def ce_group(base_l, acc, pstart=pstart, cb=cb):
      base_g = pstart + base_l
      pidx = base_g + iota
      valid = pidx < P
      fidx = (base_l + iota) * NUM_CLASSES
      xs = [plsc.load_gather(cb, [fidx + j]) for j in range(NUM_CLASSES)]
      m = _maxtree(xs)
      lse = _ln(_sumtree([jnp.exp(x - m) for x in xs])) + m
      ctg = ct_r[pl.ds(base_g, L)]
      tgt = plsc.load_gather(cb, [fidx + ctg])
      ce = lse - tgt
      pos = ctg > 0
      acc = acc + jnp.where(pos & valid, ce, 0.0)
      vv_r[pl.ds(base_g, L)] = jnp.where(pos | (~valid), 0.0, ce)
      return acc

    if cnt == CHUNK_P:
      def g4(g, acc):
        acc = ce_group(g * (2 * L), acc)
        return ce_group(g * (2 * L) + L, acc)

      sp_acc = lax.fori_loop(0, ngr // 2, g4, sp_acc)
    else:
      def g4t(g, acc):
        return ce_group(g * L, acc)

      sp_acc = lax.fori_loop(0, ngr, g4t, sp_acc)
  vv_r[pl.ds(NFULL * CHUNK_P + ((TAIL_P + L - 1) // L) * L,
             PPAD - NFULL * CHUNK_P - ((TAIL_P + L - 1) // L) * L)] = (
      jnp.zeros((PPAD - NFULL * CHUNK_P - ((TAIL_P + L - 1) // L) * L,)))

  k = jnp.minimum(jnp.int32(NEGPOS_RATIO) * npos, jnp.int32(P - 1))

  ones = jnp.ones((L,), jnp.int32)
  prefix = jnp.int32(0)
  kk = k
  for shift, width in ((22, 9), (13, 9), (4, 9), (0, 4)):
    nbins = 1 << width
    ngrp = nbins // L

    def zh(g, _):
      hist[pl.ds(g * L, L)] = jnp.zeros((L,), jnp.int32)
      return 0

    lax.fori_loop(0, ngrp, zh, 0)
    pref_v = jnp.full((L,), prefix)
    lo_mask = jnp.int32(nbins - 1)

    def hp(g, _, shift=shift, width=width, pref_v=pref_v, lo_mask=lo_mask):
      vb = plsc.bitcast(vv_r[pl.ds(g * L, L)], jnp.int32)
      m = lax.shift_right_logical(vb, shift + width) == pref_v
      b = lax.shift_right_logical(vb, shift) & lo_mask
      plsc.addupdate_scatter(hist, [b], ones, mask=m)
      return 0

    lax.fori_loop(0, NG, hp, 0)

    if ngrp > 1:
      h2[pl.ds(0, L)] = jnp.zeros((L,), jnp.int32)
      h2[pl.ds(L, L)] = jnp.zeros((L,), jnp.int32)

      def gtot(gi, _):
        plsc.addupdate_scatter(h2, [jnp.full((L,), gi)],
                               hist[pl.ds(gi * L, L)])
        return 0

      lax.fori_loop(0, ngrp, gtot, 0)

      def sg2(i, carry, ngrp=ngrp):
        cum, fg, krem = carry
        gi2 = ngrp // L - 1 - i
        hv = h2[pl.ds(gi2 * L, L)]
        for lane in range(L - 1, -1, -1):
          c = hv[lane]
          ncum = cum + c
          hit = (ncum >= kk) & (cum < kk)
          fg = jnp.where(hit, gi2 * L + lane, fg)
          krem = jnp.where(hit, kk - cum, krem)
          cum = ncum
        return cum, fg, krem

      cum_above, fg, kk2 = lax.fori_loop(
          0, ngrp // L, sg2, (jnp.int32(0), jnp.int32(0), kk))
      hv = hist[pl.ds(fg * L, L)]
    else:
      fg = jnp.int32(0)
      kk2 = kk
      hv = hist[pl.ds(0, L)]

    cum = jnp.int32(0)
    fl = jnp.int32(0)
    krem = kk2
    for lane in range(L - 1, -1, -1):
      c = hv[lane]
      ncum = cum + c
      hit = (ncum >= kk2) & (cum < kk2)
      fl = jnp.where(hit, jnp.int32(lane), fl)
      krem = jnp.where(hit, kk2 - cum, krem)
      cum = ncum
    fbin = fg * L + fl
    prefix = (prefix << width) | fbin
    kk = krem

  bits = jnp.where(k > 0, prefix, jnp.int32(0x7F000000))
  bstar = jnp.full((L,), bits)

  def gf(g, carry):
    sh, ch = carry
    v = vv_r[pl.ds(g * L, L)]
    vb = plsc.bitcast(v, jnp.int32)
    gt = vb > bstar
    sh = sh + jnp.where(gt, v, 0.0)
    ch = ch + jnp.where(gt, 1, 0).astype(jnp.int32)
    return sh, ch

  sh, ch = lax.fori_loop(0, NG, gf, (jnp.zeros((L,)), jnp.zeros((L,), jnp.int32)))
  sum_hi = _hsum(sh)
  c_hi = _hsum(ch)
  tie_val = plsc.bitcast(bstar, jnp.float32)[0]
  loss_c = _hsum(sp_acc) + sum_hi + (k - c_hi).astype(jnp.float32) * tie_val

  outv = jnp.where(iota == 0, jnp.full((L,), loss_l),
                   jnp.where(iota == 1, jnp.full((L,), loss_c),
                             jnp.where(iota == 2,
                                       jnp.full((L,), npos.astype(jnp.float32)),
                                       0.0)))
  res[...] = outv
  pltpu.sync_copy(res, out_h.at[img])


@jax.jit
def _run(conf, loc, pri, tgt):
  mesh = plsc.VectorSubcoreMesh(core_axis_name="c", subcore_axis_name="s",
                                num_cores=2, num_subcores=16)
  f = pl.kernel(
      _body,
      out_type=jax.ShapeDtypeStruct((B, L), jnp.float32),
      mesh=mesh,
      compiler_params=pltpu.CompilerParams(needs_layout_passes=False,
                                           use_tc_tiling_on_sc=False),
      scratch_types=[
          pltpu.VMEM((4 * PPAD,), jnp.float32),
          pltpu.VMEM((4 * PPAD,), jnp.float32),
          pltpu.VMEM((O * 5,), jnp.float32),
          pltpu.VMEM((PPAD,), jnp.float32),
          pltpu.VMEM((PPAD,), jnp.int32),
          pltpu.VMEM((PPAD,), jnp.int32),
          pltpu.VMEM((PPAD,), jnp.float32),
          pltpu.VMEM((CHUNK_P * NUM_CLASSES,), jnp.float32),
          pltpu.VMEM((CHUNK_P * NUM_CLASSES,), jnp.float32),
          pltpu.VMEM((512,), jnp.int32),
          pltpu.VMEM((2 * L,), jnp.int32),
          pltpu.VMEM((L,), jnp.float32),
          pltpu.SemaphoreType.DMA,
          pltpu.SemaphoreType.DMA,
      ],
  )
  return f(conf, loc, pri, tgt)


def kernel(arm_loc_data, arm_conf_data, odm_loc_data, odm_conf_data,
           priors, targets):
  del odm_loc_data, odm_conf_data
  one = lax.optimization_barrier(jnp.float32(1.0))
  out = _run(arm_conf_data.reshape(B, P * NUM_CLASSES) * one,
             arm_loc_data.reshape(B, P * 4) * one,
             priors.reshape(P * 4),
             targets.reshape(B, O * 5))
  loss_l = jnp.sum(out[:, 0])
  loss_c = jnp.sum(out[:, 1])
  n = jnp.sum(out[:, 2])
  return (loss_l / n, loss_c / n)


# speedup vs baseline: 1.5891x; 1.0084x over previous
import jax
import jax.numpy as jnp
from jax import lax
from jax.experimental import pallas as pl
from jax.experimental.pallas import tpu as pltpu
from jax.experimental.pallas import tpu_sc as plsc

NUM_CLASSES = 21
THRESHOLD = 0.5
NEGPOS_RATIO = 3
VAR0, VAR1 = 0.1, 0.2

B = 32
P = 6375
O = 16
L = 16
PPAD = 6400
NG = PPAD // L
CHUNK_P = 640
NFULL = P // CHUNK_P
TAIL_P = P - NFULL * CHUNK_P

_LN2 = 0.6931471805599453
_SQRT2 = 1.4142135623730951


def _hsum(v):
  s = v[0]
  for i in range(1, L):
    s = s + v[i]
  return s


def _hmax(v):
  s = v[0]
  for i in range(1, L):
    s = jnp.maximum(s, v[i])
  return s


def _hmin(v):
  s = v[0]
  for i in range(1, L):
    s = jnp.minimum(s, v[i])
  return s


def _maxtree(xs):
  xs = list(xs)
  while len(xs) > 1:
    nxt = [jnp.maximum(xs[i], xs[i + 1]) for i in range(0, len(xs) - 1, 2)]
    if len(xs) % 2:
      nxt.append(xs[-1])
    xs = nxt
  return xs[0]


def _sumtree(xs):
  xs = list(xs)
  while len(xs) > 1:
    nxt = [xs[i] + xs[i + 1] for i in range(0, len(xs) - 1, 2)]
    if len(xs) % 2:
      nxt.append(xs[-1])
    xs = nxt
  return xs[0]


def _ln(x):
  b = plsc.bitcast(x, jnp.int32)
  e = lax.shift_right_logical(b, 23) - 127
  m = plsc.bitcast((b & 0x007FFFFF) | 0x3F800000, jnp.float32)
  big = m > _SQRT2
  m = jnp.where(big, m * 0.5, m)
  e = jnp.where(big, e + 1, e)
  z = (m - 1.0) / (m + 1.0)
  z2 = z * z
  p = 2.0 + z2 * (2.0 / 3.0 + z2 * (2.0 / 5.0 + z2 * (2.0 / 7.0 + z2 * (2.0 / 9.0))))
  return e.astype(jnp.float32) * _LN2 + z * p


def _body(conf_h, loc_h, pri_h, tgt_h, out_h,
          pv, lv, tv, bto_r, bti_r, ct_r, vv_r, buf, buf2,
          hist, h2, res, sem, sem2):
  img = lax.axis_index("s") * 2 + lax.axis_index("c")
  iota = lax.iota(jnp.int32, L)

  pltpu.sync_copy(pri_h, pv)
  pltpu.sync_copy(loc_h.at[img], lv)
  pltpu.sync_copy(tgt_h.at[img], tv)

  def tcol(idx, c):
    return plsc.load_gather(tv, [idx * 5 + c])

  r0 = tcol(iota, 0)
  r1 = tcol(iota, 1)
  r2 = tcol(iota, 2)
  r3 = tcol(iota, 3)
  t_x0 = [jnp.full((L,), r0[t]) for t in range(O)]
  t_y0 = [jnp.full((L,), r1[t]) for t in range(O)]
  t_x1 = [jnp.full((L,), r2[t]) for t in range(O)]
  t_y1 = [jnp.full((L,), r3[t]) for t in range(O)]
  t_ar = [(t_x1[t] - t_x0[t]) * (t_y1[t] - t_y0[t]) for t in range(O)]

  def g1_sweep(trange, first):
    def body(g, carry):
      bv, bi = carry
      base = g * L
      pidx = base + iota
      pidc4 = jnp.minimum(pidx, P - 1) * 4
      px = plsc.load_gather(pv, [pidc4])
      py = plsc.load_gather(pv, [pidc4 + 1])
      pw = plsc.load_gather(pv, [pidc4 + 2])
      ph = plsc.load_gather(pv, [pidc4 + 3])
      x0 = px - pw * 0.5
      x1 = px + pw * 0.5
      y0 = py - ph * 0.5
      y1 = py + ph * 0.5
      area_p = pw * ph
      if first:
        bto_g = jnp.full((L,), -1.0)
        bti_g = jnp.zeros((L,), jnp.int32)
      else:
        bto_g = bto_r[pl.ds(base, L)]
        bti_g = bti_r[pl.ds(base, L)]
      nbv = list(bv)
      nbi = list(bi)
      for i, t in enumerate(trange):
        ix0 = jnp.maximum(x0, t_x0[t])
        ix1 = jnp.minimum(x1, t_x1[t])
        iy0 = jnp.maximum(y0, t_y0[t])
        iy1 = jnp.minimum(y1, t_y1[t])
        iw = jnp.maximum(ix1 - ix0, 0.0)
        ih = jnp.maximum(iy1 - iy0, 0.0)
        inter = iw * ih
        iou = inter / (t_ar[t] + area_p - inter)
        up = iou > bto_g
        bto_g = jnp.where(up, iou, bto_g)
        bti_g = jnp.where(up, t, bti_g)
        upt = iou > nbv[i]
        nbv[i] = jnp.where(upt, iou, nbv[i])
        nbi[i] = jnp.where(upt, pidx, nbi[i])
      if first:
        bto_r[pl.ds(base, L)] = bto_g
      else:
        bto_r[pl.ds(base, L)] = jnp.where(pidx < P, bto_g, 0.0)
      bti_r[pl.ds(base, L)] = bti_g
      return tuple(nbv), tuple(nbi)

    init = (tuple(jnp.full((L,), -2.0) for _ in trange),
            tuple(jnp.zeros((L,), jnp.int32) for _ in trange))
    return lax.fori_loop(0, NG, body, init)

  bvf0, bif0 = g1_sweep(range(0, 8), True)
  bvf1, bif1 = g1_sweep(range(8, O), False)
  bvf = tuple(bvf0) + tuple(bvf1)
  bif = tuple(bif0) + tuple(bif1)

  bpiv = jnp.zeros((L,), jnp.int32)
  for t in range(O):
    m = _hmax(bvf[t])
    cand = jnp.where(bvf[t] == m, bif[t], jnp.int32(P))
    bpiv = jnp.where(iota == t, jnp.full((L,), _hmin(cand)), bpiv)

  plsc.store_scatter(bto_r, [bpiv], jnp.full((L,), 2.0))
  plsc.store_scatter(bti_r, [bpiv], iota)

  def g3(g, carry):
    ll, npv = carry
    base = g * L
    pidc4 = jnp.minimum(base + iota, P - 1) * 4
    btog = bto_r[pl.ds(base, L)]
    btig = bti_r[pl.ds(base, L)]
    pos = btog >= THRESHOLD
    lab = tcol(btig, 4)
    ct = jnp.where(pos, lab.astype(jnp.int32) + 1, 0)
    ct_r[pl.ds(base, L)] = ct
    m0 = tcol(btig, 0)
    m1 = tcol(btig, 1)
    m2 = tcol(btig, 2)
    m3 = tcol(btig, 3)
    px = plsc.load_gather(pv, [pidc4])
    py = plsc.load_gather(pv, [pidc4 + 1])
    pw = plsc.load_gather(pv, [pidc4 + 2])
    ph = plsc.load_gather(pv, [pidc4 + 3])
    g0 = ((m0 + m2) * 0.5 - px) / (pw * VAR0)
    g1v = ((m1 + m3) * 0.5 - py) / (ph * VAR0)
    g2 = _ln((m2 - m0) / pw) * (1.0 / VAR1)
    g3v = _ln((m3 - m1) / ph) * (1.0 / VAR1)
    acc = jnp.zeros((L,))
    for c, gc in enumerate((g0, g1v, g2, g3v)):
      d = plsc.load_gather(lv, [pidc4 + c]) - gc
      ad = jnp.abs(d)
      acc = acc + jnp.where(ad < 1.0, 0.5 * d * d, ad - 0.5)
    ll = ll + jnp.where(pos, acc, 0.0)
    npv = npv + jnp.where(pos, 1, 0).astype(jnp.int32)
    return ll, npv

  ll, npv = lax.fori_loop(0, NG, g3, (jnp.zeros((L,)), jnp.zeros((L,), jnp.int32)))
  loss_l = _hsum(ll)
  npos = _hsum(npv)

  sp_acc = jnp.zeros((L,))
  bufs = (buf, buf2)
  sems = (sem, sem2)

  def start_chunk(ci):
    pstart = ci * CHUNK_P
    cnt = CHUNK_P if ci < NFULL else TAIL_P
    b = bufs[ci % 2]
    dst = b if ci < NFULL else b.at[pl.ds(0, TAIL_P * NUM_CLASSES)]
    return pltpu.async_copy(
        conf_h.at[img, pl.ds(pstart * NUM_CLASSES, cnt * NUM_CLASSES)], dst,
        sems[ci % 2])

  descs = [start_chunk(0)]
  for ci in range(NFULL + 1):
    if ci + 1 <= NFULL:
      descs.append(start_chunk(ci + 1))
    descs[ci].wait()
    pstart = ci * CHUNK_P
    cnt = CHUNK_P if ci < NFULL else TAIL_P
    cb = bufs[ci % 2]
    ngr = (cnt + L - 1) // L

    def ce_group(base_l, acc, pstart=pstart, cb=cb):
      base_g = pstart + base_l
      pidx = base_g + iota
      valid = pidx < P
      fidx = (base_l + iota) * NUM_CLASSES
      xs = [plsc.load_gather(cb, [fidx + j]) for j in range(NUM_CLASSES)]
      m = _maxtree(xs)
      lse = _ln(_sumtree([jnp.exp(x - m) for x in xs])) + m
      ctg = ct_r[pl.ds(base_g, L)]
      tgt = plsc.load_gather(cb, [fidx + ctg])
      ce = lse - tgt
      pos = ctg > 0
      acc = acc + jnp.where(pos & valid, ce, 0.0)
      vv_r[pl.ds(base_g, L)] = jnp.where(pos | (~valid), 0.0, ce)
      return acc

    if cnt == CHUNK_P:
      def g4(g, acc):
        acc = ce_group(g * (2 * L), acc)
        return ce_group(g * (2 * L) + L, acc)

      sp_acc = lax.fori_loop(0, ngr // 2, g4, sp_acc)
    else:
      def g4t(g, acc):
        return ce_group(g * L, acc)

      sp_acc = lax.fori_loop(0, ngr, g4t, sp_acc)
  vv_r[pl.ds(NFULL * CHUNK_P + ((TAIL_P + L - 1) // L) * L,
             PPAD - NFULL * CHUNK_P - ((TAIL_P + L - 1) // L) * L)] = (
      jnp.zeros((PPAD - NFULL * CHUNK_P - ((TAIL_P + L - 1) // L) * L,)))

  k = jnp.minimum(jnp.int32(NEGPOS_RATIO) * npos, jnp.int32(P - 1))

  ones = jnp.ones((L,), jnp.int32)
  prefix = jnp.int32(0)
  kk = k
  for shift, width in ((22, 9), (13, 9), (4, 9), (0, 4)):
    nbins = 1 << width
    ngrp = nbins // L

    def zh(g, _):
      hist[pl.ds(g * L, L)] = jnp.zeros((L,), jnp.int32)
      return 0

    lax.fori_loop(0, ngrp, zh, 0)
    pref_v = jnp.full((L,), prefix)
    lo_mask = jnp.int32(nbins - 1)

    def hp(g, _, shift=shift, width=width, pref_v=pref_v, lo_mask=lo_mask):
      vb = plsc.bitcast(vv_r[pl.ds(g * L, L)], jnp.int32)
      m = lax.shift_right_logical(vb, shift + width) == pref_v
      b = lax.shift_right_logical(vb, shift) & lo_mask
      plsc.addupdate_scatter(hist, [b], ones, mask=m)
      return 0

    lax.fori_loop(0, NG, hp, 0)

    if ngrp > 1:
      h2[pl.ds(0, L)] = jnp.zeros((L,), jnp.int32)
      h2[pl.ds(L, L)] = jnp.zeros((L,), jnp.int32)

      def gtot(gi, _):
        plsc.addupdate_scatter(h2, [jnp.full((L,), gi)],
                               hist[pl.ds(gi * L, L)])
        return 0

      lax.fori_loop(0, ngrp, gtot, 0)

      def sg2(i, carry, ngrp=ngrp):
        cum, fg, krem = carry
        gi2 = ngrp // L - 1 - i
        hv = h2[pl.ds(gi2 * L, L)]
        for lane in range(L - 1, -1, -1):
          c = hv[lane]
          ncum = cum + c
          hit = (ncum >= kk) & (cum < kk)
          fg = jnp.where(hit, gi2 * L + lane, fg)
          krem = jnp.where(hit, kk - cum, krem)
          cum = ncum
        return cum, fg, krem

      cum_above, fg, kk2 = lax.fori_loop(
          0, ngrp // L, sg2, (jnp.int32(0), jnp.int32(0), kk))
      hv = hist[pl.ds(fg * L, L)]
    else:
      fg = jnp.int32(0)
      kk2 = kk
      hv = hist[pl.ds(0, L)]

    cum = jnp.int32(0)
    fl = jnp.int32(0)
    krem = kk2
    for lane in range(L - 1, -1, -1):
      c = hv[lane]
      ncum = cum + c
      hit = (ncum >= kk2) & (cum < kk2)
      fl = jnp.where(hit, jnp.int32(lane), fl)
      krem = jnp.where(hit, kk2 - cum, krem)
      cum = ncum
    fbin = fg * L + fl
    prefix = (prefix << width) | fbin
    kk = krem

  bits = jnp.where(k > 0, prefix, jnp.int32(0x7F000000))
  bstar = jnp.full((L,), bits)

  def gf(g, carry):
    sh, ch = carry
    v = vv_r[pl.ds(g * L, L)]
    vb = plsc.bitcast(v, jnp.int32)
    gt = vb > bstar
    sh = sh + jnp.where(gt, v, 0.0)
    ch = ch + jnp.where(gt, 1, 0).astype(jnp.int32)
    return sh, ch

  sh, ch = lax.fori_loop(0, NG, gf, (jnp.zeros((L,)), jnp.zeros((L,), jnp.int32)))
  sum_hi = _hsum(sh)
  c_hi = _hsum(ch)
  tie_val = plsc.bitcast(bstar, jnp.float32)[0]
  loss_c = _hsum(sp_acc) + sum_hi + (k - c_hi).astype(jnp.float32) * tie_val

  outv = jnp.where(iota == 0, jnp.full((L,), loss_l),
                   jnp.where(iota == 1, jnp.full((L,), loss_c),
                             jnp.where(iota == 2,
                                       jnp.full((L,), npos.astype(jnp.float32)),
                                       0.0)))
  res[...] = outv
  pltpu.sync_copy(res, out_h.at[img])


@jax.jit
def _run(conf, loc, pri, tgt):
  mesh = plsc.VectorSubcoreMesh(core_axis_name="c", subcore_axis_name="s",
                                num_cores=2, num_subcores=16)
  f = pl.kernel(
      _body,
      out_type=jax.ShapeDtypeStruct((B, L), jnp.float32),
      mesh=mesh,
      compiler_params=pltpu.CompilerParams(needs_layout_passes=False,
                                           use_tc_tiling_on_sc=False),
      scratch_types=[
          pltpu.VMEM((P * 4,), jnp.float32),
          pltpu.VMEM((P * 4,), jnp.float32),
          pltpu.VMEM((O * 5,), jnp.float32),
          pltpu.VMEM((PPAD,), jnp.float32),
          pltpu.VMEM((PPAD,), jnp.int32),
          pltpu.VMEM((PPAD,), jnp.int32),
          pltpu.VMEM((PPAD,), jnp.float32),
          pltpu.VMEM((CHUNK_P * NUM_CLASSES,), jnp.float32),
          pltpu.VMEM((CHUNK_P * NUM_CLASSES,), jnp.float32),
          pltpu.VMEM((512,), jnp.int32),
          pltpu.VMEM((2 * L,), jnp.int32),
          pltpu.VMEM((L,), jnp.float32),
          pltpu.SemaphoreType.DMA,
          pltpu.SemaphoreType.DMA,
      ],
  )
  return f(conf, loc, pri, tgt)


def kernel(arm_loc_data, arm_conf_data, odm_loc_data, odm_conf_data,
           priors, targets):
  del odm_loc_data, odm_conf_data
  one = lax.optimization_barrier(jnp.float32(1.0))
  out = _run(arm_conf_data.reshape(B, P * NUM_CLASSES) * one,
             arm_loc_data.reshape(B, P * 4) * one,
             priors.reshape(P * 4),
             targets.reshape(B, O * 5))
  loss_l = jnp.sum(out[:, 0])
  loss_c = jnp.sum(out[:, 1])
  n = jnp.sum(out[:, 2])
  return (loss_l / n, loss_c / n)
